# baseline (speedup 1.0000x reference)
import functools
import math

import jax
import jax.numpy as jnp
import numpy as np
from jax import lax
from jax.experimental import pallas as pl
from jax.experimental.pallas import tpu as pltpu
from jax.experimental.pallas import tpu_sc as plsc

N = 10000
E = 320000
D = 128
NC = 2
NS = 16
NP = 2
NPC = N // NC
NPP = 2560
NPT = 160
W = 1280
NWIN = E // W
G = 16
ACC = NPT * D
NBLK = NC * NP * NS

_hist = np.array([0.0] * 32 + [10000.0], dtype=np.float64)
_bins = np.arange(_hist.shape[0], dtype=np.float64)
_AVG_LOG = float((np.log(_bins + 1.0) * _hist).sum() / _hist.sum())

_ROWS = 2000


def _ab_body(x_ref, w1_ref, w2_ref, pb_ref, a_ref, b_ref):
    xv = x_ref[...]
    a_ref[...] = jnp.dot(xv, w1_ref[...],
                         preferred_element_type=jnp.float32,
                         precision=lax.Precision.HIGHEST) + pb_ref[...]
    b_ref[...] = jnp.dot(xv, w2_ref[...], preferred_element_type=jnp.float32,
                         precision=lax.Precision.HIGHEST)


def _ab_matmul(x, w1, w2, pb):
    return pl.pallas_call(
        _ab_body,
        grid=(N // _ROWS,),
        in_specs=[
            pl.BlockSpec((_ROWS, D), lambda i: (i, 0)),
            pl.BlockSpec((D, D), lambda i: (0, 0)),
            pl.BlockSpec((D, D), lambda i: (0, 0)),
            pl.BlockSpec((1, D), lambda i: (0, 0)),
        ],
        out_specs=[
            pl.BlockSpec((_ROWS, D), lambda i: (i, 0)),
            pl.BlockSpec((_ROWS, D), lambda i: (i, 0)),
        ],
        out_shape=[
            jax.ShapeDtypeStruct((N, D), jnp.float32),
            jax.ShapeDtypeStruct((N, D), jnp.float32),
        ],
    )(x, w1, w2, pb)


def _post_body(a_ref, s_ref, q_ref, m_ref, w_ref, x_ref, cnt_ref, pw_ref,
               pb_ref, lw_ref, lb_ref, o_ref):
    a = a_ref[...]
    s = s_ref[...]
    cnt = cnt_ref[...]
    degc = jnp.maximum(cnt, 1.0)
    inv = 1.0 / degc
    mean = (cnt * a + s) * inv
    msq = (cnt * (a * a) + 2.0 * a * s + q_ref[...]) * inv
    var = msq - mean * mean
    std = jnp.sqrt(jnp.maximum(var, 0.0) + 1e-5)
    has = cnt > 0
    mx = jnp.where(has, a + m_ref[...], 0.0)
    mn = jnp.where(has, a + w_ref[...], 0.0)
    scl = jnp.log(degc + 1.0) * (1.0 / _AVG_LOG)
    inv_scl = 1.0 / scl

    def mm(v, r0):
        return jnp.dot(v, pw_ref[r0 * D:(r0 + 1) * D, :],
                       preferred_element_type=jnp.float32)

    h = mm(x_ref[...], 0)
    h += mm(mean, 1) + mm(mn, 2) + mm(mx, 3) + mm(std, 4)
    h += mm(mean * scl, 5) + mm(mn * scl, 6) + mm(mx * scl, 7) + mm(std * scl, 8)
    h += (mm(mean * inv_scl, 9) + mm(mn * inv_scl, 10) + mm(mx * inv_scl, 11)
          + mm(std * inv_scl, 12))
    h += pb_ref[...]
    o_ref[...] = jnp.dot(h, lw_ref[...],
                         preferred_element_type=jnp.float32) + lb_ref[...]


def _post_matmul(a, s, q, m, w, x, cnt, pw, pb, lw, lb):
    row = lambda i: (i, 0)
    fix = lambda i: (0, 0)
    return pl.pallas_call(
        _post_body,
        grid=(N // _ROWS,),
        in_specs=[
            pl.BlockSpec((_ROWS, D), row),
            pl.BlockSpec((_ROWS, D), row),
            pl.BlockSpec((_ROWS, D), row),
            pl.BlockSpec((_ROWS, D), row),
            pl.BlockSpec((_ROWS, D), row),
            pl.BlockSpec((_ROWS, D), row),
            pl.BlockSpec((_ROWS, 1), row),
            pl.BlockSpec((13 * D, D), fix),
            pl.BlockSpec((1, D), fix),
            pl.BlockSpec((D, D), fix),
            pl.BlockSpec((1, D), fix),
        ],
        out_specs=pl.BlockSpec((_ROWS, D), row),
        out_shape=jax.ShapeDtypeStruct((N, D), jnp.float32),
    )(a, s, q, m, w, x, cnt, pw, pb, lw, lb)


def _sc_body(b_hbm, src_hbm, dst_hbm, minit_hbm, winit_hbm,
             s_out, q_out, m_out, w_out, cnt_out,
             gbuf, macc, wacc, sacc, qacc, cntacc, stag_src, stag_dloc,
             dstwin, srcwin, sem, sem_st):
    c = lax.axis_index("c")
    s = lax.axis_index("s")
    lanes = lax.iota(jnp.int32, 16)

    def zero_f32(ref, n16):
        def body(t, _):
            ref[pl.ds(t * 16, 16)] = jnp.zeros((16,), jnp.float32)
            return 0
        lax.fori_loop(0, n16, body, 0)

    def zero_i32(ref, n16):
        def body(t, _):
            ref[pl.ds(t * 16, 16)] = jnp.zeros((16,), jnp.int32)
            return 0
        lax.fori_loop(0, n16, body, 0)

    for p in range(NP):
        base = c * NPC + p * NPP + s * NPT
        hi = jnp.minimum(base + NPT, (c + 1) * NPC)

        pltpu.sync_copy(minit_hbm, macc)
        pltpu.sync_copy(winit_hbm, wacc)
        zero_f32(sacc, ACC // 16)
        zero_f32(qacc, ACC // 16)
        zero_f32(cntacc, NPT // 16)
        zero_i32(stag_src, W // 16)

        def stage_win(wi, par):
            pltpu.async_copy(dst_hbm.at[pl.ds(wi * W, W)],
                             dstwin.at[par], sem_st)
            pltpu.async_copy(src_hbm.at[pl.ds(wi * W, W)],
                             srcwin.at[par], sem_st)

        def wait_win(wi, par):
            pltpu.make_async_copy(dst_hbm.at[pl.ds(wi * W, W)],
                                  dstwin.at[par], sem_st).wait()
            pltpu.make_async_copy(src_hbm.at[pl.ds(wi * W, W)],
                                  srcwin.at[par], sem_st).wait()

        def issue_gather(j, h):
            pltpu.async_copy(b_hbm.at[stag_src.at[pl.ds(j * G, G)]],
                             gbuf.at[pl.ds(h * G, G)], sem)

        def wait_gather(j, h):
            pltpu.make_async_copy(b_hbm.at[stag_src.at[pl.ds(j * G, G)]],
                                  gbuf.at[pl.ds(h * G, G)], sem).wait()

        stage_win(0, 0)

        def window(wi, _):
            par = wi & 1
            wait_win(wi, par)

            @pl.when(wi + 1 < NWIN)
            def _():
                stage_win(wi + 1, 1 - par)

            def fbody(c4, kc):
                for u in range(4):
                    ch = c4 * 4 + u
                    d = dstwin[par, pl.ds(ch * 16, 16)]
                    sv = srcwin[par, pl.ds(ch * 16, 16)]
                    msk = (d >= base) & (d < hi)
                    csum = plsc.cumsum(jnp.where(msk, 1, 0))
                    pos = kc + csum - 1
                    plsc.store_scatter(stag_src, [pos], sv, mask=msk)
                    plsc.store_scatter(stag_dloc, [pos], (d - base) * D,
                                       mask=msk)
                    kc = kc + plsc.all_reduce_population_count(msk)
                return kc

            kvec = lax.fori_loop(0, W // 64, fbody,
                                 jnp.zeros((16,), jnp.int32))
            k = jnp.max(kvec)
            nch = (k + G - 1) // G

            @pl.when(nch > 0)
            def _():
                issue_gather(0, 0)

            def gchunk(j, _):
                h = j & 1
                wait_gather(j, h)

                @pl.when(j + 1 < nch)
                def _():
                    issue_gather(j + 1, 1 - h)

                ne = jnp.minimum(G, k - j * G)
                dlocs = stag_dloc[pl.ds(j * G, G)]
                row0 = h * G

                def ebody(i, _):
                    dl = jnp.sum(jnp.where(lanes == i, dlocs, 0))
                    for cc in range(D // 16):
                        bv = gbuf[row0 + i, pl.ds(cc * 16, 16)]
                        off = dl + cc * 16
                        mo = macc[pl.ds(off, 16)]
                        macc[pl.ds(off, 16)] = jnp.maximum(mo, bv)
                        wo = wacc[pl.ds(off, 16)]
                        wacc[pl.ds(off, 16)] = jnp.minimum(wo, bv)
                        so = sacc[pl.ds(off, 16)]
                        sacc[pl.ds(off, 16)] = so + bv
                        qo = qacc[pl.ds(off, 16)]
                        qacc[pl.ds(off, 16)] = qo + bv * bv
                    loc = lax.shift_right_logical(dl, 7)
                    coff = lax.shift_left(lax.shift_right_logical(loc, 4), 4)
                    lane = loc & 15
                    cv = cntacc[pl.ds(coff, 16)]
                    cntacc[pl.ds(coff, 16)] = cv + jnp.where(
                        lanes == lane, 1.0, 0.0)
                    return 0

                lax.fori_loop(0, ne, ebody, 0)
                return 0

            lax.fori_loop(0, nch, gchunk, 0)
            return 0

        lax.fori_loop(0, NWIN, window, 0)

        blk = (c * NP + p) * NS + s
        pltpu.sync_copy(sacc, s_out.at[blk])
        pltpu.sync_copy(qacc, q_out.at[blk])
        pltpu.sync_copy(macc, m_out.at[blk])
        pltpu.sync_copy(wacc, w_out.at[blk])
        pltpu.sync_copy(cntacc, cnt_out.at[blk])


def _sc_segment(b, src, dst, minit, winit):
    mesh = plsc.VectorSubcoreMesh(core_axis_name="c", subcore_axis_name="s")
    f = pl.kernel(
        _sc_body,
        out_type=[
            jax.ShapeDtypeStruct((NBLK, ACC), jnp.float32),
            jax.ShapeDtypeStruct((NBLK, ACC), jnp.float32),
            jax.ShapeDtypeStruct((NBLK, ACC), jnp.float32),
            jax.ShapeDtypeStruct((NBLK, ACC), jnp.float32),
            jax.ShapeDtypeStruct((NBLK, NPT), jnp.float32),
        ],
        mesh=mesh,
        compiler_params=pltpu.CompilerParams(needs_layout_passes=False),
        scratch_types=[
            pltpu.VMEM((2 * G, D), jnp.float32),
            pltpu.VMEM((ACC,), jnp.float32),
            pltpu.VMEM((ACC,), jnp.float32),
            pltpu.VMEM((ACC,), jnp.float32),
            pltpu.VMEM((ACC,), jnp.float32),
            pltpu.VMEM((NPT,), jnp.float32),
            pltpu.VMEM((W,), jnp.int32),
            pltpu.VMEM((W,), jnp.int32),
            pltpu.VMEM((2, W), jnp.int32),
            pltpu.VMEM((2, W), jnp.int32),
            pltpu.SemaphoreType.DMA,
            pltpu.SemaphoreType.DMA,
        ],
    )
    return f(b, src, dst, minit, winit)


def _perms():
    n = np.arange(N)
    cidx = n // NPC
    r = n - cidx * NPC
    p = r // NPP
    r2 = r - p * NPP
    sidx = r2 // NPT
    loc = r2 - sidx * NPT
    perm = ((cidx * NP + p) * NS + sidx) * NPT + loc
    return perm.astype(np.int32)


_PERM_MW = _perms()


def _bn_relu(o, g, b):
    mu = jnp.mean(o, axis=0)
    var = jnp.var(o, axis=0)
    return jax.nn.relu((o - mu) / jnp.sqrt(var + 1e-5) * g + b)


def kernel(x, edge_index, pre_w0, pre_b0, post_w0, post_b0, lin_w0, lin_b0,
           bn_g0, bn_b0, pre_w1, pre_b1, post_w1, post_b1, lin_w1, lin_b1,
           bn_g1, bn_b1):
    src = edge_index[0]
    dst = edge_index[1]
    minit = jnp.full((ACC,), -1e30, jnp.float32)
    winit = jnp.full((ACC,), 1e30, jnp.float32)
    pmw = jnp.asarray(_PERM_MW)

    def layer(o, pw, pb, ow, ob, lw, lb):
        a, b = _ab_matmul(o, pw[:D], pw[D:], pb[None, :])
        s_raw, q_raw, m_raw, w_raw, c_raw = _sc_segment(b, src, dst, minit,
                                                        winit)
        ss = jnp.take(s_raw.reshape(NBLK * NPT, D), pmw, axis=0)
        qq = jnp.take(q_raw.reshape(NBLK * NPT, D), pmw, axis=0)
        mm = jnp.take(m_raw.reshape(NBLK * NPT, D), pmw, axis=0)
        ww = jnp.take(w_raw.reshape(NBLK * NPT, D), pmw, axis=0)
        cnt = jnp.take(c_raw.reshape(NBLK * NPT), pmw)[:, None]
        return _post_matmul(a, ss, qq, mm, ww, o, cnt,
                            ow, ob[None, :], lw, lb[None, :])

    o = x
    hs = [o]
    o = layer(o, pre_w0, pre_b0, post_w0, post_b0, lin_w0, lin_b0)
    o = _bn_relu(o, bn_g0, bn_b0)
    hs.append(o)
    o = layer(o, pre_w1, pre_b1, post_w1, post_b1, lin_w1, lin_b1)
    o = _bn_relu(o, bn_g1, bn_b1)
    hs.append(o)
    return jnp.concatenate(hs, axis=1)


# speedup vs baseline: 1.1677x; 1.1677x over previous
import functools
import math

import jax
import jax.numpy as jnp
import numpy as np
from jax import lax
from jax.experimental import pallas as pl
from jax.experimental.pallas import tpu as pltpu
from jax.experimental.pallas import tpu_sc as plsc

N = 10000
E = 320000
D = 128
NC = 2
NS = 16
NP = 2
NPC = N // NC
NPP = 2560
NPT = 160
W = 3200
NWIN = E // W
G = 32
ACC = NPT * D
NBLK = NC * NP * NS

_hist = np.array([0.0] * 32 + [10000.0], dtype=np.float64)
_bins = np.arange(_hist.shape[0], dtype=np.float64)
_AVG_LOG = float((np.log(_bins + 1.0) * _hist).sum() / _hist.sum())

_ROWS = 2000


def _ab_body(x_ref, w1_ref, w2_ref, pb_ref, a_ref, b_ref):
    xv = x_ref[...]
    a_ref[...] = jnp.dot(xv, w1_ref[...],
                         preferred_element_type=jnp.float32,
                         precision=lax.Precision.HIGHEST) + pb_ref[...]
    b_ref[...] = jnp.dot(xv, w2_ref[...], preferred_element_type=jnp.float32,
                         precision=lax.Precision.HIGHEST)


def _ab_matmul(x, w1, w2, pb):
    return pl.pallas_call(
        _ab_body,
        grid=(N // _ROWS,),
        in_specs=[
            pl.BlockSpec((_ROWS, D), lambda i: (i, 0)),
            pl.BlockSpec((D, D), lambda i: (0, 0)),
            pl.BlockSpec((D, D), lambda i: (0, 0)),
            pl.BlockSpec((1, D), lambda i: (0, 0)),
        ],
        out_specs=[
            pl.BlockSpec((_ROWS, D), lambda i: (i, 0)),
            pl.BlockSpec((_ROWS, D), lambda i: (i, 0)),
        ],
        out_shape=[
            jax.ShapeDtypeStruct((N, D), jnp.float32),
            jax.ShapeDtypeStruct((N, D), jnp.float32),
        ],
    )(x, w1, w2, pb)


def _post_body(a_ref, s_ref, q_ref, m_ref, w_ref, x_ref, cnt_ref, pw_ref,
               pb_ref, lw_ref, lb_ref, o_ref):
    a = a_ref[...]
    s = s_ref[...]
    cnt = cnt_ref[...]
    degc = jnp.maximum(cnt, 1.0)
    inv = 1.0 / degc
    mean = (cnt * a + s) * inv
    msq = (cnt * (a * a) + 2.0 * a * s + q_ref[...]) * inv
    var = msq - mean * mean
    std = jnp.sqrt(jnp.maximum(var, 0.0) + 1e-5)
    has = cnt > 0
    mx = jnp.where(has, a + m_ref[...], 0.0)
    mn = jnp.where(has, a + w_ref[...], 0.0)
    scl = jnp.log(degc + 1.0) * (1.0 / _AVG_LOG)
    inv_scl = 1.0 / scl

    def mm(v, r0):
        return jnp.dot(v, pw_ref[r0 * D:(r0 + 1) * D, :],
                       preferred_element_type=jnp.float32)

    h = mm(x_ref[...], 0)
    h += mm(mean, 1) + mm(mn, 2) + mm(mx, 3) + mm(std, 4)
    h += mm(mean * scl, 5) + mm(mn * scl, 6) + mm(mx * scl, 7) + mm(std * scl, 8)
    h += (mm(mean * inv_scl, 9) + mm(mn * inv_scl, 10) + mm(mx * inv_scl, 11)
          + mm(std * inv_scl, 12))
    h += pb_ref[...]
    o_ref[...] = jnp.dot(h, lw_ref[...],
                         preferred_element_type=jnp.float32) + lb_ref[...]


def _post_matmul(a, s, q, m, w, x, cnt, pw, pb, lw, lb):
    row = lambda i: (i, 0)
    fix = lambda i: (0, 0)
    return pl.pallas_call(
        _post_body,
        grid=(N // _ROWS,),
        in_specs=[
            pl.BlockSpec((_ROWS, D), row),
            pl.BlockSpec((_ROWS, D), row),
            pl.BlockSpec((_ROWS, D), row),
            pl.BlockSpec((_ROWS, D), row),
            pl.BlockSpec((_ROWS, D), row),
            pl.BlockSpec((_ROWS, D), row),
            pl.BlockSpec((_ROWS, 1), row),
            pl.BlockSpec((13 * D, D), fix),
            pl.BlockSpec((1, D), fix),
            pl.BlockSpec((D, D), fix),
            pl.BlockSpec((1, D), fix),
        ],
        out_specs=pl.BlockSpec((_ROWS, D), row),
        out_shape=jax.ShapeDtypeStruct((N, D), jnp.float32),
    )(a, s, q, m, w, x, cnt, pw, pb, lw, lb)


def _sc_body(b_hbm, ei_hbm, minit_hbm, winit_hbm,
             s_out, q_out, m_out, w_out, cnt_out,
             gbuf, macc, wacc, sacc, qacc, cntacc, stag_src, stag_dloc,
             winbuf, sem, sem_st):
    c = lax.axis_index("c")
    s = lax.axis_index("s")
    lanes = lax.iota(jnp.int32, 16)

    def zero_f32(ref, n16):
        def body(t, _):
            ref[pl.ds(t * 16, 16)] = jnp.zeros((16,), jnp.float32)
            return 0
        lax.fori_loop(0, n16, body, 0)

    def zero_i32(ref, n16):
        def body(t, _):
            ref[pl.ds(t * 16, 16)] = jnp.zeros((16,), jnp.int32)
            return 0
        lax.fori_loop(0, n16, body, 0)

    for p in range(NP):
        base = c * NPC + p * NPP + s * NPT
        hi = jnp.minimum(base + NPT, (c + 1) * NPC)

        pltpu.sync_copy(minit_hbm, macc)
        pltpu.sync_copy(winit_hbm, wacc)
        zero_f32(sacc, ACC // 16)
        zero_f32(qacc, ACC // 16)
        zero_f32(cntacc, NPT // 16)
        zero_i32(stag_src, W // 16)

        def stage_win(wi, par):
            pltpu.async_copy(ei_hbm.at[:, pl.ds(wi * W, W)],
                             winbuf.at[par], sem_st)

        def wait_win(wi, par):
            pltpu.make_async_copy(ei_hbm.at[:, pl.ds(wi * W, W)],
                                  winbuf.at[par], sem_st).wait()

        def issue_gather(j, h):
            pltpu.async_copy(b_hbm.at[stag_src.at[pl.ds(j * G, G)]],
                             gbuf.at[pl.ds(h * G, G)], sem)

        def wait_gather(j, h):
            pltpu.make_async_copy(b_hbm.at[stag_src.at[pl.ds(j * G, G)]],
                                  gbuf.at[pl.ds(h * G, G)], sem).wait()

        stage_win(0, 0)

        def window(wi, _):
            par = wi & 1
            wait_win(wi, par)

            @pl.when(wi + 1 < NWIN)
            def _():
                stage_win(wi + 1, 1 - par)

            def fbody(c4, kc):
                for u in range(4):
                    ch = c4 * 4 + u
                    d = winbuf[par, 1, pl.ds(ch * 16, 16)]
                    sv = winbuf[par, 0, pl.ds(ch * 16, 16)]
                    msk = (d >= base) & (d < hi)
                    csum = plsc.cumsum(jnp.where(msk, 1, 0))
                    pos = kc + csum - 1
                    plsc.store_scatter(stag_src, [pos], sv, mask=msk)
                    plsc.store_scatter(stag_dloc, [pos], (d - base) * D,
                                       mask=msk)
                    kc = kc + plsc.all_reduce_population_count(msk)
                return kc

            kvec = lax.fori_loop(0, W // 64, fbody,
                                 jnp.zeros((16,), jnp.int32))
            k = jnp.max(kvec)
            nch = (k + G - 1) // G

            @pl.when(nch > 0)
            def _():
                issue_gather(0, 0)

            def gchunk(j, _):
                h = j & 1
                wait_gather(j, h)

                @pl.when(j + 1 < nch)
                def _():
                    issue_gather(j + 1, 1 - h)

                ne = jnp.minimum(G, k - j * G)
                dlo = stag_dloc[pl.ds(j * G, 16)]
                dhi = stag_dloc[pl.ds(j * G + 16, 16)]
                row0 = h * G

                def ebody(i, _):
                    dl = (jnp.sum(jnp.where(lanes == i, dlo, 0))
                          + jnp.sum(jnp.where(lanes == i - 16, dhi, 0)))
                    for cc in range(D // 16):
                        bv = gbuf[row0 + i, pl.ds(cc * 16, 16)]
                        off = dl + cc * 16
                        mo = macc[pl.ds(off, 16)]
                        macc[pl.ds(off, 16)] = jnp.maximum(mo, bv)
                        wo = wacc[pl.ds(off, 16)]
                        wacc[pl.ds(off, 16)] = jnp.minimum(wo, bv)
                        so = sacc[pl.ds(off, 16)]
                        sacc[pl.ds(off, 16)] = so + bv
                        qo = qacc[pl.ds(off, 16)]
                        qacc[pl.ds(off, 16)] = qo + bv * bv
                    loc = lax.shift_right_logical(dl, 7)
                    coff = lax.shift_left(lax.shift_right_logical(loc, 4), 4)
                    lane = loc & 15
                    cv = cntacc[pl.ds(coff, 16)]
                    cntacc[pl.ds(coff, 16)] = cv + jnp.where(
                        lanes == lane, 1.0, 0.0)
                    return 0

                lax.fori_loop(0, ne, ebody, 0)
                return 0

            lax.fori_loop(0, nch, gchunk, 0)
            return 0

        lax.fori_loop(0, NWIN, window, 0)

        blk = (c * NP + p) * NS + s
        pltpu.sync_copy(sacc, s_out.at[blk])
        pltpu.sync_copy(qacc, q_out.at[blk])
        pltpu.sync_copy(macc, m_out.at[blk])
        pltpu.sync_copy(wacc, w_out.at[blk])
        pltpu.sync_copy(cntacc, cnt_out.at[blk])


def _sc_segment(b, ei, minit, winit):
    mesh = plsc.VectorSubcoreMesh(core_axis_name="c", subcore_axis_name="s")
    f = pl.kernel(
        _sc_body,
        out_type=[
            jax.ShapeDtypeStruct((NBLK, ACC), jnp.float32),
            jax.ShapeDtypeStruct((NBLK, ACC), jnp.float32),
            jax.ShapeDtypeStruct((NBLK, ACC), jnp.float32),
            jax.ShapeDtypeStruct((NBLK, ACC), jnp.float32),
            jax.ShapeDtypeStruct((NBLK, NPT), jnp.float32),
        ],
        mesh=mesh,
        compiler_params=pltpu.CompilerParams(needs_layout_passes=False),
        scratch_types=[
            pltpu.VMEM((2 * G, D), jnp.float32),
            pltpu.VMEM((ACC,), jnp.float32),
            pltpu.VMEM((ACC,), jnp.float32),
            pltpu.VMEM((ACC,), jnp.float32),
            pltpu.VMEM((ACC,), jnp.float32),
            pltpu.VMEM((NPT,), jnp.float32),
            pltpu.VMEM((W,), jnp.int32),
            pltpu.VMEM((W,), jnp.int32),
            pltpu.VMEM((2, 2, W), jnp.int32),
            pltpu.SemaphoreType.DMA,
            pltpu.SemaphoreType.DMA,
        ],
    )
    return f(b, ei, minit, winit)


def _perms():
    n = np.arange(N)
    cidx = n // NPC
    r = n - cidx * NPC
    p = r // NPP
    r2 = r - p * NPP
    sidx = r2 // NPT
    loc = r2 - sidx * NPT
    perm = ((cidx * NP + p) * NS + sidx) * NPT + loc
    return perm.astype(np.int32)


_PERM_MW = _perms()


def _bn_relu(o, g, b):
    mu = jnp.mean(o, axis=0)
    var = jnp.var(o, axis=0)
    return jax.nn.relu((o - mu) / jnp.sqrt(var + 1e-5) * g + b)


def kernel(x, edge_index, pre_w0, pre_b0, post_w0, post_b0, lin_w0, lin_b0,
           bn_g0, bn_b0, pre_w1, pre_b1, post_w1, post_b1, lin_w1, lin_b1,
           bn_g1, bn_b1):
    minit = jnp.full((ACC,), -1e30, jnp.float32)
    winit = jnp.full((ACC,), 1e30, jnp.float32)
    pmw = jnp.asarray(_PERM_MW)

    def layer(o, pw, pb, ow, ob, lw, lb):
        a, b = _ab_matmul(o, pw[:D], pw[D:], pb[None, :])
        s_raw, q_raw, m_raw, w_raw, c_raw = _sc_segment(b, edge_index, minit,
                                                        winit)
        ss = jnp.take(s_raw.reshape(NBLK * NPT, D), pmw, axis=0)
        qq = jnp.take(q_raw.reshape(NBLK * NPT, D), pmw, axis=0)
        mm = jnp.take(m_raw.reshape(NBLK * NPT, D), pmw, axis=0)
        ww = jnp.take(w_raw.reshape(NBLK * NPT, D), pmw, axis=0)
        cnt = jnp.take(c_raw.reshape(NBLK * NPT), pmw)[:, None]
        return _post_matmul(a, ss, qq, mm, ww, o, cnt,
                            ow, ob[None, :], lw, lb[None, :])

    o = x
    hs = [o]
    o = layer(o, pre_w0, pre_b0, post_w0, post_b0, lin_w0, lin_b0)
    o = _bn_relu(o, bn_g0, bn_b0)
    hs.append(o)
    o = layer(o, pre_w1, pre_b1, post_w1, post_b1, lin_w1, lin_b1)
    o = _bn_relu(o, bn_g1, bn_b1)
    hs.append(o)
    return jnp.concatenate(hs, axis=1)


# speedup vs baseline: 1.2241x; 1.0483x over previous
import functools
import math

import jax
import jax.numpy as jnp
import numpy as np
from jax import lax
from jax.experimental import pallas as pl
from jax.experimental.pallas import tpu as pltpu
from jax.experimental.pallas import tpu_sc as plsc

N = 10000
E = 320000
D = 128
NC = 2
NS = 16
NP = 2
NPC = N // NC
NPP = 2560
NPT = 160
W = 6400
NWIN = E // W
G = 32
ACC = NPT * D
NBLK = NC * NP * NS

_hist = np.array([0.0] * 32 + [10000.0], dtype=np.float64)
_bins = np.arange(_hist.shape[0], dtype=np.float64)
_AVG_LOG = float((np.log(_bins + 1.0) * _hist).sum() / _hist.sum())

_ROWS = 2000


def _ab_body(x_ref, w1_ref, w2_ref, pb_ref, a_ref, b_ref):
    xv = x_ref[...]
    a_ref[...] = jnp.dot(xv, w1_ref[...],
                         preferred_element_type=jnp.float32,
                         precision=lax.Precision.HIGHEST) + pb_ref[...]
    b_ref[...] = jnp.dot(xv, w2_ref[...], preferred_element_type=jnp.float32,
                         precision=lax.Precision.HIGHEST)


def _ab_matmul(x, w1, w2, pb):
    return pl.pallas_call(
        _ab_body,
        grid=(N // _ROWS,),
        in_specs=[
            pl.BlockSpec((_ROWS, D), lambda i: (i, 0)),
            pl.BlockSpec((D, D), lambda i: (0, 0)),
            pl.BlockSpec((D, D), lambda i: (0, 0)),
            pl.BlockSpec((1, D), lambda i: (0, 0)),
        ],
        out_specs=[
            pl.BlockSpec((_ROWS, D), lambda i: (i, 0)),
            pl.BlockSpec((_ROWS, D), lambda i: (i, 0)),
        ],
        out_shape=[
            jax.ShapeDtypeStruct((N, D), jnp.float32),
            jax.ShapeDtypeStruct((N, D), jnp.float32),
        ],
    )(x, w1, w2, pb)


def _post_body(a_ref, s_ref, q_ref, m_ref, w_ref, x_ref, cnt_ref, pw_ref,
               pb_ref, lw_ref, lb_ref, o_ref):
    a = a_ref[...]
    s = s_ref[...]
    cnt = cnt_ref[...]
    degc = jnp.maximum(cnt, 1.0)
    inv = 1.0 / degc
    mean = (cnt * a + s) * inv
    msq = (cnt * (a * a) + 2.0 * a * s + q_ref[...]) * inv
    var = msq - mean * mean
    std = jnp.sqrt(jnp.maximum(var, 0.0) + 1e-5)
    has = cnt > 0
    mx = jnp.where(has, a + m_ref[...], 0.0)
    mn = jnp.where(has, a + w_ref[...], 0.0)
    scl = jnp.log(degc + 1.0) * (1.0 / _AVG_LOG)
    inv_scl = 1.0 / scl

    def mm(v, r0):
        return jnp.dot(v, pw_ref[r0 * D:(r0 + 1) * D, :],
                       preferred_element_type=jnp.float32)

    h = mm(x_ref[...], 0)
    h += mm(mean, 1) + mm(mn, 2) + mm(mx, 3) + mm(std, 4)
    h += mm(mean * scl, 5) + mm(mn * scl, 6) + mm(mx * scl, 7) + mm(std * scl, 8)
    h += (mm(mean * inv_scl, 9) + mm(mn * inv_scl, 10) + mm(mx * inv_scl, 11)
          + mm(std * inv_scl, 12))
    h += pb_ref[...]
    o_ref[...] = jnp.dot(h, lw_ref[...],
                         preferred_element_type=jnp.float32) + lb_ref[...]


def _post_matmul(a, s, q, m, w, x, cnt, pw, pb, lw, lb):
    row = lambda i: (i, 0)
    fix = lambda i: (0, 0)
    return pl.pallas_call(
        _post_body,
        grid=(N // _ROWS,),
        in_specs=[
            pl.BlockSpec((_ROWS, D), row),
            pl.BlockSpec((_ROWS, D), row),
            pl.BlockSpec((_ROWS, D), row),
            pl.BlockSpec((_ROWS, D), row),
            pl.BlockSpec((_ROWS, D), row),
            pl.BlockSpec((_ROWS, D), row),
            pl.BlockSpec((_ROWS, 1), row),
            pl.BlockSpec((13 * D, D), fix),
            pl.BlockSpec((1, D), fix),
            pl.BlockSpec((D, D), fix),
            pl.BlockSpec((1, D), fix),
        ],
        out_specs=pl.BlockSpec((_ROWS, D), row),
        out_shape=jax.ShapeDtypeStruct((N, D), jnp.float32),
    )(a, s, q, m, w, x, cnt, pw, pb, lw, lb)


def _sc_body(b_hbm, ei_hbm, minit_hbm, winit_hbm,
             s_out, q_out, m_out, w_out, cnt_out,
             gbuf, macc, wacc, sacc, qacc, cntacc, stag_src, stag_dloc,
             winbuf, sem, sem_st):
    c = lax.axis_index("c")
    s = lax.axis_index("s")
    lanes = lax.iota(jnp.int32, 16)

    def zero_f32(ref, n16):
        def body(t, _):
            ref[pl.ds(t * 16, 16)] = jnp.zeros((16,), jnp.float32)
            return 0
        lax.fori_loop(0, n16, body, 0)

    def zero_i32(ref, n16):
        def body(t, _):
            ref[pl.ds(t * 16, 16)] = jnp.zeros((16,), jnp.int32)
            return 0
        lax.fori_loop(0, n16, body, 0)

    for p in range(NP):
        base = c * NPC + p * NPP + s * NPT
        hi = jnp.minimum(base + NPT, (c + 1) * NPC)

        pltpu.sync_copy(minit_hbm, macc)
        pltpu.sync_copy(winit_hbm, wacc)
        zero_f32(sacc, ACC // 16)
        zero_f32(qacc, ACC // 16)
        zero_f32(cntacc, NPT // 16)
        zero_i32(stag_src, W // 16)

        def stage_win(wi, par):
            pltpu.async_copy(ei_hbm.at[:, pl.ds(wi * W, W)],
                             winbuf.at[par], sem_st)

        def wait_win(wi, par):
            pltpu.make_async_copy(ei_hbm.at[:, pl.ds(wi * W, W)],
                                  winbuf.at[par], sem_st).wait()

        def issue_gather(j, h):
            pltpu.async_copy(b_hbm.at[stag_src.at[pl.ds(j * G, G)]],
                             gbuf.at[pl.ds(h * G, G)], sem)

        def wait_gather(j, h):
            pltpu.make_async_copy(b_hbm.at[stag_src.at[pl.ds(j * G, G)]],
                                  gbuf.at[pl.ds(h * G, G)], sem).wait()

        stage_win(0, 0)

        def window(wi, _):
            par = wi & 1
            wait_win(wi, par)

            @pl.when(wi + 1 < NWIN)
            def _():
                stage_win(wi + 1, 1 - par)

            def fbody(c4, kc):
                for u in range(4):
                    ch = c4 * 4 + u
                    d = winbuf[par, 1, pl.ds(ch * 16, 16)]
                    sv = winbuf[par, 0, pl.ds(ch * 16, 16)]
                    msk = (d >= base) & (d < hi)
                    csum = plsc.cumsum(jnp.where(msk, 1, 0))
                    pos = kc + csum - 1
                    plsc.store_scatter(stag_src, [pos], sv, mask=msk)
                    plsc.store_scatter(stag_dloc, [pos], (d - base) * D,
                                       mask=msk)
                    kc = kc + plsc.all_reduce_population_count(msk)
                return kc

            kvec = lax.fori_loop(0, W // 64, fbody,
                                 jnp.zeros((16,), jnp.int32))
            k = jnp.max(kvec)
            nch = (k + G - 1) // G

            @pl.when(nch > 0)
            def _():
                issue_gather(0, 0)

            def gchunk(j, _):
                h = j & 1
                wait_gather(j, h)

                @pl.when(j + 1 < nch)
                def _():
                    issue_gather(j + 1, 1 - h)

                ne = jnp.minimum(G, k - j * G)
                dlo = stag_dloc[pl.ds(j * G, 16)]
                dhi = stag_dloc[pl.ds(j * G + 16, 16)]
                row0 = h * G

                def ebody(i, _):
                    dl = (jnp.sum(jnp.where(lanes == i, dlo, 0))
                          + jnp.sum(jnp.where(lanes == i - 16, dhi, 0)))
                    for cc in range(D // 16):
                        bv = gbuf[row0 + i, pl.ds(cc * 16, 16)]
                        off = dl + cc * 16
                        mo = macc[pl.ds(off, 16)]
                        macc[pl.ds(off, 16)] = jnp.maximum(mo, bv)
                        wo = wacc[pl.ds(off, 16)]
                        wacc[pl.ds(off, 16)] = jnp.minimum(wo, bv)
                        so = sacc[pl.ds(off, 16)]
                        sacc[pl.ds(off, 16)] = so + bv
                        qo = qacc[pl.ds(off, 16)]
                        qacc[pl.ds(off, 16)] = qo + bv * bv
                    loc = lax.shift_right_logical(dl, 7)
                    coff = lax.shift_left(lax.shift_right_logical(loc, 4), 4)
                    lane = loc & 15
                    cv = cntacc[pl.ds(coff, 16)]
                    cntacc[pl.ds(coff, 16)] = cv + jnp.where(
                        lanes == lane, 1.0, 0.0)
                    return 0

                lax.fori_loop(0, ne, ebody, 0)
                return 0

            lax.fori_loop(0, nch, gchunk, 0)
            return 0

        lax.fori_loop(0, NWIN, window, 0)

        blk = (c * NP + p) * NS + s
        pltpu.sync_copy(sacc, s_out.at[blk])
        pltpu.sync_copy(qacc, q_out.at[blk])
        pltpu.sync_copy(macc, m_out.at[blk])
        pltpu.sync_copy(wacc, w_out.at[blk])
        pltpu.sync_copy(cntacc, cnt_out.at[blk])


def _sc_segment(b, ei, minit, winit):
    mesh = plsc.VectorSubcoreMesh(core_axis_name="c", subcore_axis_name="s")
    f = pl.kernel(
        _sc_body,
        out_type=[
            jax.ShapeDtypeStruct((NBLK, ACC), jnp.float32),
            jax.ShapeDtypeStruct((NBLK, ACC), jnp.float32),
            jax.ShapeDtypeStruct((NBLK, ACC), jnp.float32),
            jax.ShapeDtypeStruct((NBLK, ACC), jnp.float32),
            jax.ShapeDtypeStruct((NBLK, NPT), jnp.float32),
        ],
        mesh=mesh,
        compiler_params=pltpu.CompilerParams(needs_layout_passes=False),
        scratch_types=[
            pltpu.VMEM((2 * G, D), jnp.float32),
            pltpu.VMEM((ACC,), jnp.float32),
            pltpu.VMEM((ACC,), jnp.float32),
            pltpu.VMEM((ACC,), jnp.float32),
            pltpu.VMEM((ACC,), jnp.float32),
            pltpu.VMEM((NPT,), jnp.float32),
            pltpu.VMEM((W,), jnp.int32),
            pltpu.VMEM((W,), jnp.int32),
            pltpu.VMEM((2, 2, W), jnp.int32),
            pltpu.SemaphoreType.DMA,
            pltpu.SemaphoreType.DMA,
        ],
    )
    return f(b, ei, minit, winit)


def _perms():
    n = np.arange(N)
    cidx = n // NPC
    r = n - cidx * NPC
    p = r // NPP
    r2 = r - p * NPP
    sidx = r2 // NPT
    loc = r2 - sidx * NPT
    perm = ((cidx * NP + p) * NS + sidx) * NPT + loc
    return perm.astype(np.int32)


_PERM_MW = _perms()


def _bn_relu(o, g, b):
    mu = jnp.mean(o, axis=0)
    var = jnp.var(o, axis=0)
    return jax.nn.relu((o - mu) / jnp.sqrt(var + 1e-5) * g + b)


def kernel(x, edge_index, pre_w0, pre_b0, post_w0, post_b0, lin_w0, lin_b0,
           bn_g0, bn_b0, pre_w1, pre_b1, post_w1, post_b1, lin_w1, lin_b1,
           bn_g1, bn_b1):
    minit = jnp.full((ACC,), -1e30, jnp.float32)
    winit = jnp.full((ACC,), 1e30, jnp.float32)
    pmw = jnp.asarray(_PERM_MW)

    def layer(o, pw, pb, ow, ob, lw, lb):
        a, b = _ab_matmul(o, pw[:D], pw[D:], pb[None, :])
        s_raw, q_raw, m_raw, w_raw, c_raw = _sc_segment(b, edge_index, minit,
                                                        winit)
        ss = jnp.take(s_raw.reshape(NBLK * NPT, D), pmw, axis=0)
        qq = jnp.take(q_raw.reshape(NBLK * NPT, D), pmw, axis=0)
        mm = jnp.take(m_raw.reshape(NBLK * NPT, D), pmw, axis=0)
        ww = jnp.take(w_raw.reshape(NBLK * NPT, D), pmw, axis=0)
        cnt = jnp.take(c_raw.reshape(NBLK * NPT), pmw)[:, None]
        return _post_matmul(a, ss, qq, mm, ww, o, cnt,
                            ow, ob[None, :], lw, lb[None, :])

    o = x
    hs = [o]
    o = layer(o, pre_w0, pre_b0, post_w0, post_b0, lin_w0, lin_b0)
    o = _bn_relu(o, bn_g0, bn_b0)
    hs.append(o)
    o = layer(o, pre_w1, pre_b1, post_w1, post_b1, lin_w1, lin_b1)
    o = _bn_relu(o, bn_g1, bn_b1)
    hs.append(o)
    return jnp.concatenate(hs, axis=1)


# speedup vs baseline: 1.2361x; 1.0098x over previous
import functools
import math

import jax
import jax.numpy as jnp
import numpy as np
from jax import lax
from jax.experimental import pallas as pl
from jax.experimental.pallas import tpu as pltpu
from jax.experimental.pallas import tpu_sc as plsc

N = 10000
E = 320000
D = 128
NC = 2
NS = 16
NP = 2
NPC = N // NC
NPP = 2560
NPT = 160
W = 6400
NWIN = E // W
G = 32
ACC = NPT * D
NBLK = NC * NP * NS

_hist = np.array([0.0] * 32 + [10000.0], dtype=np.float64)
_bins = np.arange(_hist.shape[0], dtype=np.float64)
_AVG_LOG = float((np.log(_bins + 1.0) * _hist).sum() / _hist.sum())

_ROWS = 2000


def _ab_body(x_ref, w1_ref, w2_ref, pb_ref, a_ref, b_ref):
    xv = x_ref[...]
    a_ref[...] = jnp.dot(xv, w1_ref[...],
                         preferred_element_type=jnp.float32,
                         precision=lax.Precision.HIGHEST) + pb_ref[...]
    b_ref[...] = jnp.dot(xv, w2_ref[...], preferred_element_type=jnp.float32,
                         precision=lax.Precision.HIGHEST)


def _ab_matmul(x, w1, w2, pb):
    return pl.pallas_call(
        _ab_body,
        grid=(N // _ROWS,),
        in_specs=[
            pl.BlockSpec((_ROWS, D), lambda i: (i, 0)),
            pl.BlockSpec((D, D), lambda i: (0, 0)),
            pl.BlockSpec((D, D), lambda i: (0, 0)),
            pl.BlockSpec((1, D), lambda i: (0, 0)),
        ],
        out_specs=[
            pl.BlockSpec((_ROWS, D), lambda i: (i, 0)),
            pl.BlockSpec((_ROWS, D), lambda i: (i, 0)),
        ],
        out_shape=[
            jax.ShapeDtypeStruct((N, D), jnp.float32),
            jax.ShapeDtypeStruct((N, D), jnp.float32),
        ],
    )(x, w1, w2, pb)


def _post_body(a_ref, s_ref, q_ref, m_ref, w_ref, x_ref, cnt_ref, pw_ref,
               pb_ref, lw_ref, lb_ref, o_ref):
    a = a_ref[...]
    s = s_ref[...]
    cnt = cnt_ref[...]
    degc = jnp.maximum(cnt, 1.0)
    inv = 1.0 / degc
    mean = (cnt * a + s) * inv
    msq = (cnt * (a * a) + 2.0 * a * s + q_ref[...]) * inv
    var = msq - mean * mean
    std = jnp.sqrt(jnp.maximum(var, 0.0) + 1e-5)
    has = cnt > 0
    mx = jnp.where(has, a + m_ref[...], 0.0)
    mn = jnp.where(has, a + w_ref[...], 0.0)
    scl = jnp.log(degc + 1.0) * (1.0 / _AVG_LOG)
    inv_scl = 1.0 / scl

    def mm(v, r0):
        return jnp.dot(v, pw_ref[r0 * D:(r0 + 1) * D, :],
                       preferred_element_type=jnp.float32)

    h = mm(x_ref[...], 0)
    h += mm(mean, 1) + mm(mn, 2) + mm(mx, 3) + mm(std, 4)
    h += mm(mean * scl, 5) + mm(mn * scl, 6) + mm(mx * scl, 7) + mm(std * scl, 8)
    h += (mm(mean * inv_scl, 9) + mm(mn * inv_scl, 10) + mm(mx * inv_scl, 11)
          + mm(std * inv_scl, 12))
    h += pb_ref[...]
    o_ref[...] = jnp.dot(h, lw_ref[...],
                         preferred_element_type=jnp.float32) + lb_ref[...]


def _post_matmul(a, s, q, m, w, x, cnt, pw, pb, lw, lb):
    row = lambda i: (i, 0)
    fix = lambda i: (0, 0)
    return pl.pallas_call(
        _post_body,
        grid=(N // _ROWS,),
        in_specs=[
            pl.BlockSpec((_ROWS, D), row),
            pl.BlockSpec((_ROWS, D), row),
            pl.BlockSpec((_ROWS, D), row),
            pl.BlockSpec((_ROWS, D), row),
            pl.BlockSpec((_ROWS, D), row),
            pl.BlockSpec((_ROWS, D), row),
            pl.BlockSpec((_ROWS, 1), row),
            pl.BlockSpec((13 * D, D), fix),
            pl.BlockSpec((1, D), fix),
            pl.BlockSpec((D, D), fix),
            pl.BlockSpec((1, D), fix),
        ],
        out_specs=pl.BlockSpec((_ROWS, D), row),
        out_shape=jax.ShapeDtypeStruct((N, D), jnp.float32),
    )(a, s, q, m, w, x, cnt, pw, pb, lw, lb)


def _sc_body(b_hbm, ei_hbm, minit_hbm, winit_hbm,
             s_out, q_out, m_out, w_out, cnt_out,
             gbuf, macc, wacc, sacc, qacc, cntacc, stag_src, stag_dloc,
             winbuf, sem, sem_st):
    c = lax.axis_index("c")
    s = lax.axis_index("s")
    lanes = lax.iota(jnp.int32, 16)

    def zero_f32(ref, n16):
        def body(t, _):
            ref[pl.ds(t * 16, 16)] = jnp.zeros((16,), jnp.float32)
            return 0
        lax.fori_loop(0, n16, body, 0)

    def zero_i32(ref, n16):
        def body(t, _):
            ref[pl.ds(t * 16, 16)] = jnp.zeros((16,), jnp.int32)
            return 0
        lax.fori_loop(0, n16, body, 0)

    for p in range(NP):
        base = c * NPC + p * NPP + s * NPT
        hi = jnp.minimum(base + NPT, (c + 1) * NPC)

        pltpu.sync_copy(minit_hbm, macc)
        pltpu.sync_copy(winit_hbm, wacc)
        zero_f32(sacc, ACC // 16)
        zero_f32(qacc, ACC // 16)
        zero_f32(cntacc, NPT // 16)
        zero_i32(stag_src, W // 16)

        def stage_win(wi, par):
            pltpu.async_copy(ei_hbm.at[:, pl.ds(wi * W, W)],
                             winbuf.at[par], sem_st)

        def wait_win(wi, par):
            pltpu.make_async_copy(ei_hbm.at[:, pl.ds(wi * W, W)],
                                  winbuf.at[par], sem_st).wait()

        def issue_gather(j, h):
            pltpu.async_copy(b_hbm.at[stag_src.at[pl.ds(j * G, G)]],
                             gbuf.at[pl.ds(h * G, G)], sem)

        def wait_gather(j, h):
            pltpu.make_async_copy(b_hbm.at[stag_src.at[pl.ds(j * G, G)]],
                                  gbuf.at[pl.ds(h * G, G)], sem).wait()

        stage_win(0, 0)

        def window(wi, _):
            par = wi & 1
            wait_win(wi, par)

            @pl.when(wi + 1 < NWIN)
            def _():
                stage_win(wi + 1, 1 - par)

            def fbody(c4, kc):
                for u in range(8):
                    ch = c4 * 8 + u
                    d = winbuf[par, 1, pl.ds(ch * 16, 16)]
                    sv = winbuf[par, 0, pl.ds(ch * 16, 16)]
                    msk = (d >= base) & (d < hi)
                    csum = plsc.cumsum(jnp.where(msk, 1, 0))
                    pos = kc + csum - 1
                    plsc.store_scatter(stag_src, [pos], sv, mask=msk)
                    plsc.store_scatter(stag_dloc, [pos], (d - base) * D,
                                       mask=msk)
                    kc = kc + plsc.all_reduce_population_count(msk)
                return kc

            kvec = lax.fori_loop(0, W // 128, fbody,
                                 jnp.zeros((16,), jnp.int32))
            k = jnp.max(kvec)
            nch = (k + G - 1) // G

            @pl.when(nch > 0)
            def _():
                issue_gather(0, 0)

            def gchunk(j, _):
                h = j & 1
                wait_gather(j, h)

                @pl.when(j + 1 < nch)
                def _():
                    issue_gather(j + 1, 1 - h)

                ne = jnp.minimum(G, k - j * G)
                dlo = stag_dloc[pl.ds(j * G, 16)]
                dhi = stag_dloc[pl.ds(j * G + 16, 16)]
                row0 = h * G

                def ebody(i, _):
                    dl = (jnp.sum(jnp.where(lanes == i, dlo, 0))
                          + jnp.sum(jnp.where(lanes == i - 16, dhi, 0)))
                    for cc in range(D // 16):
                        bv = gbuf[row0 + i, pl.ds(cc * 16, 16)]
                        off = dl + cc * 16
                        mo = macc[pl.ds(off, 16)]
                        macc[pl.ds(off, 16)] = jnp.maximum(mo, bv)
                        wo = wacc[pl.ds(off, 16)]
                        wacc[pl.ds(off, 16)] = jnp.minimum(wo, bv)
                        so = sacc[pl.ds(off, 16)]
                        sacc[pl.ds(off, 16)] = so + bv
                        qo = qacc[pl.ds(off, 16)]
                        qacc[pl.ds(off, 16)] = qo + bv * bv
                    loc = lax.shift_right_logical(dl, 7)
                    coff = lax.shift_left(lax.shift_right_logical(loc, 4), 4)
                    lane = loc & 15
                    cv = cntacc[pl.ds(coff, 16)]
                    cntacc[pl.ds(coff, 16)] = cv + jnp.where(
                        lanes == lane, 1.0, 0.0)
                    return 0

                lax.fori_loop(0, ne, ebody, 0)
                return 0

            lax.fori_loop(0, nch, gchunk, 0)
            return 0

        lax.fori_loop(0, NWIN, window, 0)

        blk = (c * NP + p) * NS + s
        pltpu.sync_copy(sacc, s_out.at[blk])
        pltpu.sync_copy(qacc, q_out.at[blk])
        pltpu.sync_copy(macc, m_out.at[blk])
        pltpu.sync_copy(wacc, w_out.at[blk])
        pltpu.sync_copy(cntacc, cnt_out.at[blk])


def _sc_segment(b, ei, minit, winit):
    mesh = plsc.VectorSubcoreMesh(core_axis_name="c", subcore_axis_name="s")
    f = pl.kernel(
        _sc_body,
        out_type=[
            jax.ShapeDtypeStruct((NBLK, ACC), jnp.float32),
            jax.ShapeDtypeStruct((NBLK, ACC), jnp.float32),
            jax.ShapeDtypeStruct((NBLK, ACC), jnp.float32),
            jax.ShapeDtypeStruct((NBLK, ACC), jnp.float32),
            jax.ShapeDtypeStruct((NBLK, NPT), jnp.float32),
        ],
        mesh=mesh,
        compiler_params=pltpu.CompilerParams(needs_layout_passes=False),
        scratch_types=[
            pltpu.VMEM((2 * G, D), jnp.float32),
            pltpu.VMEM((ACC,), jnp.float32),
            pltpu.VMEM((ACC,), jnp.float32),
            pltpu.VMEM((ACC,), jnp.float32),
            pltpu.VMEM((ACC,), jnp.float32),
            pltpu.VMEM((NPT,), jnp.float32),
            pltpu.VMEM((W,), jnp.int32),
            pltpu.VMEM((W,), jnp.int32),
            pltpu.VMEM((2, 2, W), jnp.int32),
            pltpu.SemaphoreType.DMA,
            pltpu.SemaphoreType.DMA,
        ],
    )
    return f(b, ei, minit, winit)


def _perms():
    n = np.arange(N)
    cidx = n // NPC
    r = n - cidx * NPC
    p = r // NPP
    r2 = r - p * NPP
    sidx = r2 // NPT
    loc = r2 - sidx * NPT
    perm = ((cidx * NP + p) * NS + sidx) * NPT + loc
    return perm.astype(np.int32)


_PERM_MW = _perms()


def _bn_relu(o, g, b):
    mu = jnp.mean(o, axis=0)
    var = jnp.var(o, axis=0)
    return jax.nn.relu((o - mu) / jnp.sqrt(var + 1e-5) * g + b)


def kernel(x, edge_index, pre_w0, pre_b0, post_w0, post_b0, lin_w0, lin_b0,
           bn_g0, bn_b0, pre_w1, pre_b1, post_w1, post_b1, lin_w1, lin_b1,
           bn_g1, bn_b1):
    minit = jnp.full((ACC,), -1e30, jnp.float32)
    winit = jnp.full((ACC,), 1e30, jnp.float32)
    pmw = jnp.asarray(_PERM_MW)

    def layer(o, pw, pb, ow, ob, lw, lb):
        a, b = _ab_matmul(o, pw[:D], pw[D:], pb[None, :])
        s_raw, q_raw, m_raw, w_raw, c_raw = _sc_segment(b, edge_index, minit,
                                                        winit)
        ss = jnp.take(s_raw.reshape(NBLK * NPT, D), pmw, axis=0)
        qq = jnp.take(q_raw.reshape(NBLK * NPT, D), pmw, axis=0)
        mm = jnp.take(m_raw.reshape(NBLK * NPT, D), pmw, axis=0)
        ww = jnp.take(w_raw.reshape(NBLK * NPT, D), pmw, axis=0)
        cnt = jnp.take(c_raw.reshape(NBLK * NPT), pmw)[:, None]
        return _post_matmul(a, ss, qq, mm, ww, o, cnt,
                            ow, ob[None, :], lw, lb[None, :])

    o = x
    hs = [o]
    o = layer(o, pre_w0, pre_b0, post_w0, post_b0, lin_w0, lin_b0)
    o = _bn_relu(o, bn_g0, bn_b0)
    hs.append(o)
    o = layer(o, pre_w1, pre_b1, post_w1, post_b1, lin_w1, lin_b1)
    o = _bn_relu(o, bn_g1, bn_b1)
    hs.append(o)
    return jnp.concatenate(hs, axis=1)
